# baseline (speedup 1.0000x reference)
import functools

import jax
import jax.numpy as jnp
from jax import lax
from jax.experimental import pallas as pl
from jax.experimental.pallas import tpu as pltpu
from jax.experimental.pallas import tpu_sc as plsc

_R = 256


_D = 6


def _sim_topk_kernel(nrows_ref, nall_ref, erows_ref, Wp_ref, bp_ref,
                     nd_ref, idx_ref, topo_ref, dist_ref, dv_ref, cv_ref,
                     ci_ref, *, kk, s):
    i = pl.program_id(1)
    nj = s // 128
    nrows = nrows_ref[0]
    nall = nall_ref[0]
    sim = jax.lax.dot_general(nrows, nall, (((1,), (1,)), ((), ())),
                              preferred_element_type=jnp.float32)
    col = jax.lax.broadcasted_iota(jnp.int32, (_R, s), 1)
    row_gid = i * _R + jax.lax.broadcasted_iota(jnp.int32, (_R, s), 0)
    dist = 1.0 - sim
    dist = jnp.where(col == row_gid, jnp.inf, dist)
    dv_ref[...] = dist

    lane = jax.lax.broadcasted_iota(jnp.int32, (_R, 128), 1)

    m6v = None
    for r in range(_D):
        m = dv_ref[:, 0:128]
        for j in range(1, nj):
            m = jnp.minimum(m, dv_ref[:, j * 128:(j + 1) * 128])
        jp = jnp.full((_R, 128), nj, dtype=jnp.int32)
        for j in range(nj - 1, -1, -1):
            jp = jnp.where(dv_ref[:, j * 128:(j + 1) * 128] == m, j, jp)
        cv_ref[:, r * 128:(r + 1) * 128] = m
        ci_ref[:, r * 128:(r + 1) * 128] = jp * 128 + lane
        for j in range(nj):
            sl = slice(j * 128, (j + 1) * 128)
            dv_ref[:, sl] = jnp.where(jp == j, jnp.inf, dv_ref[:, sl])
        if r == _D - 1:
            m6v = m

    big = jnp.int32(1 << 30)

    def body(j, _):
        cv = cv_ref[...]
        ci = ci_ref[...]
        m = jnp.min(cv, axis=1)
        eq = cv == m[:, None]
        cand = jnp.min(jnp.where(eq, ci, big), axis=1)
        cv_ref[...] = jnp.where(ci == cand[:, None], jnp.inf, cv)
        nd_ref[0, pl.ds(j, 1), :] = m[None, :]
        idx_ref[0, pl.ds(j, 1), :] = cand[None, :]
        return 0

    jax.lax.fori_loop(0, kk, body, 0)

    nd31 = nd_ref[0, pl.ds(kk - 1, 1), :]
    fb = jnp.any(m6v <= nd31[0][:, None])

    @pl.when(fb)
    def _fallback():
        sim2 = jax.lax.dot_general(nrows_ref[0], nall_ref[0], (((1,), (1,)), ((), ())),
                                   preferred_element_type=jnp.float32)
        d2 = 1.0 - sim2
        dist_ref[...] = jnp.where(col == row_gid, jnp.inf, d2)

        def fbody(j, _):
            d = dist_ref[...]
            m = jnp.min(d, axis=1)
            eq = d == m[:, None]
            cand = jnp.min(jnp.where(eq, col, s), axis=1)
            dist_ref[...] = jnp.where(col == cand[:, None], jnp.inf, d)
            nd_ref[0, pl.ds(j, 1), :] = m[None, :]
            idx_ref[0, pl.ds(j, 1), :] = cand[None, :]
            return 0

        jax.lax.fori_loop(0, kk, fbody, 0)

    topo_ref[0] = jax.lax.dot_general(
        erows_ref[0], Wp_ref[...], (((1,), (0,)), ((), ())),
        preferred_element_type=jnp.float32) + bp_ref[...][None, :]


def _sim_topk(norm, emb, Wp, bp, kk):
    b, s, e = emb.shape
    t = Wp.shape[1]
    grid = (b, s // _R)
    nd_t, idx_t, topo = pl.pallas_call(
        functools.partial(_sim_topk_kernel, kk=kk, s=s),
        grid=grid,
        in_specs=[
            pl.BlockSpec((1, _R, e), lambda bi, i: (bi, i, 0)),
            pl.BlockSpec((1, s, e), lambda bi, i: (bi, 0, 0)),
            pl.BlockSpec((1, _R, e), lambda bi, i: (bi, i, 0)),
            pl.BlockSpec((e, t), lambda bi, i: (0, 0)),
            pl.BlockSpec((t,), lambda bi, i: (0,)),
        ],
        out_specs=[
            pl.BlockSpec((1, kk, _R), lambda bi, i: (bi, 0, i)),
            pl.BlockSpec((1, kk, _R), lambda bi, i: (bi, 0, i)),
            pl.BlockSpec((1, _R, t), lambda bi, i: (bi, i, 0)),
        ],
        out_shape=[
            jax.ShapeDtypeStruct((b, kk, s), jnp.float32),
            jax.ShapeDtypeStruct((b, kk, s), jnp.int32),
            jax.ShapeDtypeStruct((b, s, t), jnp.float32),
        ],
        scratch_shapes=[
            pltpu.VMEM((_R, s), jnp.float32),
            pltpu.VMEM((_R, s), jnp.float32),
            pltpu.VMEM((_R, _D * 128), jnp.float32),
            pltpu.VMEM((_R, _D * 128), jnp.int32),
        ],
    )(norm, norm, emb, Wp, bp)
    return (jnp.swapaxes(nd_t, 1, 2), jnp.swapaxes(idx_t, 1, 2), topo)


_RC = 256


def _combine_mlp_kernel(topo_all_ref, topo_rows_ref, nd_ref, idx_ref,
                        W1_ref, b1_ref, W2_ref, b2_ref, gamma_ref, beta_ref,
                        W3_ref, b3_ref, W4_ref, b4_ref, out_ref, *, kk, s):
    nd = nd_ref[0]
    idx = idx_ref[0]
    mneg = jnp.max(-nd, axis=1, keepdims=True)
    ew = jnp.exp(-nd - mneg)
    w = ew / jnp.sum(ew, axis=1, keepdims=True)
    col = jax.lax.broadcasted_iota(jnp.int32, (_RC, s), 1)
    A = jnp.zeros((_RC, s), dtype=jnp.float32)
    for k in range(kk):
        A = A + w[:, k:k + 1] * (col == idx[:, k:k + 1]).astype(jnp.float32)
    weighted = jax.lax.dot_general(A, topo_all_ref[0], (((1,), (0,)), ((), ())),
                                   preferred_element_type=jnp.float32)
    comb = topo_rows_ref[0] + weighted
    h = jnp.maximum(jnp.dot(comb, W1_ref[...], preferred_element_type=jnp.float32)
                    + b1_ref[...][None, :], 0.0)
    h = jnp.dot(h, W2_ref[...], preferred_element_type=jnp.float32) + b2_ref[...][None, :]
    mu = jnp.mean(h, axis=-1, keepdims=True)
    var = jnp.mean((h - mu) ** 2, axis=-1, keepdims=True)
    hn = (h - mu) / jnp.sqrt(var + 1e-5) * gamma_ref[...][None, :] + beta_ref[...][None, :]
    g = jnp.maximum(jnp.dot(hn, W3_ref[...], preferred_element_type=jnp.float32)
                    + b3_ref[...][None, :], 0.0)
    out_ref[0] = jnp.dot(g, W4_ref[...], preferred_element_type=jnp.float32) + b4_ref[...][None, :]


def _combine_mlp(topo, nd, idx, W1, b1, W2, b2, gamma, beta, W3, b3, W4, b4, kk):
    b, s, t = topo.shape
    grid = (b, s // _RC)
    out = pl.pallas_call(
        functools.partial(_combine_mlp_kernel, kk=kk, s=s),
        grid=grid,
        in_specs=[
            pl.BlockSpec((1, s, t), lambda bi, i: (bi, 0, 0)),
            pl.BlockSpec((1, _RC, t), lambda bi, i: (bi, i, 0)),
            pl.BlockSpec((1, _RC, kk), lambda bi, i: (bi, i, 0)),
            pl.BlockSpec((1, _RC, kk), lambda bi, i: (bi, i, 0)),
            pl.BlockSpec(W1.shape, lambda bi, i: (0, 0)),
            pl.BlockSpec(b1.shape, lambda bi, i: (0,)),
            pl.BlockSpec(W2.shape, lambda bi, i: (0, 0)),
            pl.BlockSpec(b2.shape, lambda bi, i: (0,)),
            pl.BlockSpec(gamma.shape, lambda bi, i: (0,)),
            pl.BlockSpec(beta.shape, lambda bi, i: (0,)),
            pl.BlockSpec(W3.shape, lambda bi, i: (0, 0)),
            pl.BlockSpec(b3.shape, lambda bi, i: (0,)),
            pl.BlockSpec(W4.shape, lambda bi, i: (0, 0)),
            pl.BlockSpec(b4.shape, lambda bi, i: (0,)),
        ],
        out_specs=pl.BlockSpec((1, _RC, t), lambda bi, i: (bi, i, 0)),
        out_shape=jax.ShapeDtypeStruct((b, s, t), jnp.float32),
    )(topo, topo, nd, idx, W1, b1, W2, b2, gamma, beta, W3, b3, W4, b4)
    return out


_CH = 2


def _sc_combine(topo_flat, gidx_flat, w_flat, kk):
    n, t = topo_flat.shape
    info = plsc.get_sparse_core_info()
    nc = info.num_cores
    nw = nc * info.num_subcores
    rows_pw = n // nw
    nchunk = rows_pw // _CH
    gpc = _CH * kk

    mesh = plsc.VectorSubcoreMesh(core_axis_name="c", subcore_axis_name="s")

    @functools.partial(
        pl.kernel, mesh=mesh,
        out_type=jax.ShapeDtypeStruct((n * t,), jnp.float32),
        scratch_types=[
            pltpu.VMEM((gpc,), jnp.int32),
            pltpu.VMEM((gpc, t), jnp.float32),
            pltpu.VMEM((gpc * 16,), jnp.float32),
            pltpu.VMEM((_CH * t,), jnp.float32),
            pltpu.SemaphoreType.DMA,
        ],
    )
    def sck(topo_hbm, gidx_hbm, w_hbm, out_hbm, idx_v, rows_v, w_v, out_v, sem):
        wid = lax.axis_index("s") * nc + lax.axis_index("c")

        def chunk_body(c, carry):
            base = wid * rows_pw + c * _CH
            off = base * kk
            pltpu.sync_copy(gidx_hbm.at[pl.ds(off, gpc)], idx_v)
            pltpu.sync_copy(w_hbm.at[pl.ds(off * 16, gpc * 16)], w_v)
            pltpu.async_copy(topo_hbm.at[idx_v], rows_v, sem).wait()
            for r in range(_CH):
                accs = [jnp.zeros((16,), jnp.float32) for _ in range(t // 16)]
                for k2 in range(kk):
                    wk = w_v[pl.ds((r * kk + k2) * 16, 16)]
                    for cc in range(t // 16):
                        accs[cc] = accs[cc] + wk * rows_v[r * kk + k2,
                                                          pl.ds(cc * 16, 16)]
                for cc in range(t // 16):
                    out_v[pl.ds(r * t + cc * 16, 16)] = accs[cc]
            pltpu.sync_copy(out_v, out_hbm.at[pl.ds(base * t, _CH * t)])
            return carry

        lax.fori_loop(0, nchunk, chunk_body, 0)

    return sck(topo_flat, gidx_flat, w_flat)


def _mlp2_kernel(topo_rows_ref, wtd_rows_ref,
                 W1_ref, b1_ref, W2_ref, b2_ref, gamma_ref, beta_ref,
                 W3_ref, b3_ref, W4_ref, b4_ref, out_ref):
    comb = topo_rows_ref[...] + wtd_rows_ref[...]
    h = jnp.maximum(jnp.dot(comb, W1_ref[...], preferred_element_type=jnp.float32)
                    + b1_ref[...][None, :], 0.0)
    h = jnp.dot(h, W2_ref[...], preferred_element_type=jnp.float32) + b2_ref[...][None, :]
    mu = jnp.mean(h, axis=-1, keepdims=True)
    var = jnp.mean((h - mu) ** 2, axis=-1, keepdims=True)
    hn = (h - mu) / jnp.sqrt(var + 1e-5) * gamma_ref[...][None, :] + beta_ref[...][None, :]
    g = jnp.maximum(jnp.dot(hn, W3_ref[...], preferred_element_type=jnp.float32)
                    + b3_ref[...][None, :], 0.0)
    out_ref[...] = jnp.dot(g, W4_ref[...], preferred_element_type=jnp.float32) + b4_ref[...][None, :]


def _mlp2(topo_flat, wtd_flat, W1, b1, W2, b2, gamma, beta, W3, b3, W4, b4):
    n, t = topo_flat.shape
    R = 1024
    out = pl.pallas_call(
        _mlp2_kernel,
        grid=(n // R,),
        in_specs=[
            pl.BlockSpec((R, t), lambda i: (i, 0)),
            pl.BlockSpec((R, t), lambda i: (i, 0)),
            pl.BlockSpec(W1.shape, lambda i: (0, 0)),
            pl.BlockSpec(b1.shape, lambda i: (0,)),
            pl.BlockSpec(W2.shape, lambda i: (0, 0)),
            pl.BlockSpec(b2.shape, lambda i: (0,)),
            pl.BlockSpec(gamma.shape, lambda i: (0,)),
            pl.BlockSpec(beta.shape, lambda i: (0,)),
            pl.BlockSpec(W3.shape, lambda i: (0, 0)),
            pl.BlockSpec(b3.shape, lambda i: (0,)),
            pl.BlockSpec(W4.shape, lambda i: (0, 0)),
            pl.BlockSpec(b4.shape, lambda i: (0,)),
        ],
        out_specs=pl.BlockSpec((R, t), lambda i: (i, 0)),
        out_shape=jax.ShapeDtypeStruct((n, t), jnp.float32),
    )(topo_flat, wtd_flat, W1, b1, W2, b2, gamma, beta, W3, b3, W4, b4)
    return out


def kernel(embeddings, Wp, bp, W1, b1, W2, b2, gamma, beta, W3, b3, W4, b4):
    b, s, e = embeddings.shape
    kk = max(1, min(32, s - 1))
    t = Wp.shape[1]
    norm = embeddings / (jnp.linalg.norm(embeddings, axis=-1, keepdims=True) + 1e-8)
    nd, idx, topo = _sim_topk(norm, embeddings, Wp, bp, kk)
    w = jax.nn.softmax(-nd, axis=-1)
    gidx = idx + (jnp.arange(b, dtype=jnp.int32) * s)[:, None, None]
    w_rep = jnp.repeat(w.reshape(b * s * kk), 16)
    wtd = _sc_combine(topo.reshape(b * s, t), gidx.reshape(b * s * kk),
                      w_rep, kk)
    p = _mlp2(topo.reshape(b * s, t), wtd.reshape(b * s, t),
              W1, b1, W2, b2, gamma, beta, W3, b3, W4, b4)
    return (p.reshape(b, s, t), nd, idx)


# speedup vs baseline: 1.5981x; 1.5981x over previous
import functools

import jax
import jax.numpy as jnp
from jax.experimental import pallas as pl
from jax.experimental.pallas import tpu as pltpu

_R = 256


_D = 6


def _sim_topk_kernel(nrows_ref, nall_ref, erows_ref, Wp_ref, bp_ref,
                     nd_ref, idx_ref, topo_ref, dist_ref, dv_ref, cv_ref,
                     ci_ref, *, kk, s):
    i = pl.program_id(1)
    nj = s // 128
    nrows = nrows_ref[0]
    nall = nall_ref[0]
    sim = jax.lax.dot_general(nrows, nall, (((1,), (1,)), ((), ())),
                              preferred_element_type=jnp.float32)
    col = jax.lax.broadcasted_iota(jnp.int32, (_R, s), 1)
    row_gid = i * _R + jax.lax.broadcasted_iota(jnp.int32, (_R, s), 0)
    dist = 1.0 - sim
    dist = jnp.where(col == row_gid, jnp.inf, dist)
    dv_ref[...] = dist

    lane = jax.lax.broadcasted_iota(jnp.int32, (_R, 128), 1)

    m6v = None
    for r in range(_D):
        m = dv_ref[:, 0:128]
        for j in range(1, nj):
            m = jnp.minimum(m, dv_ref[:, j * 128:(j + 1) * 128])
        jp = jnp.full((_R, 128), nj, dtype=jnp.int32)
        for j in range(nj - 1, -1, -1):
            jp = jnp.where(dv_ref[:, j * 128:(j + 1) * 128] == m, j, jp)
        cv_ref[:, r * 128:(r + 1) * 128] = m
        ci_ref[:, r * 128:(r + 1) * 128] = jp * 128 + lane
        for j in range(nj):
            sl = slice(j * 128, (j + 1) * 128)
            dv_ref[:, sl] = jnp.where(jp == j, jnp.inf, dv_ref[:, sl])
        if r == _D - 1:
            m6v = m

    big = jnp.int32(1 << 30)

    def body(j, _):
        cv = cv_ref[...]
        ci = ci_ref[...]
        m = jnp.min(cv, axis=1)
        eq = cv == m[:, None]
        cand = jnp.min(jnp.where(eq, ci, big), axis=1)
        cv_ref[...] = jnp.where(ci == cand[:, None], jnp.inf, cv)
        nd_ref[0, pl.ds(j, 1), :] = m[None, :]
        idx_ref[0, pl.ds(j, 1), :] = cand[None, :]
        return 0

    jax.lax.fori_loop(0, kk, body, 0)

    nd31 = nd_ref[0, pl.ds(kk - 1, 1), :]
    fb = jnp.any(m6v <= nd31[0][:, None])

    @pl.when(fb)
    def _fallback():
        sim2 = jax.lax.dot_general(nrows_ref[0], nall_ref[0], (((1,), (1,)), ((), ())),
                                   preferred_element_type=jnp.float32)
        d2 = 1.0 - sim2
        dist_ref[...] = jnp.where(col == row_gid, jnp.inf, d2)

        def fbody(j, _):
            d = dist_ref[...]
            m = jnp.min(d, axis=1)
            eq = d == m[:, None]
            cand = jnp.min(jnp.where(eq, col, s), axis=1)
            dist_ref[...] = jnp.where(col == cand[:, None], jnp.inf, d)
            nd_ref[0, pl.ds(j, 1), :] = m[None, :]
            idx_ref[0, pl.ds(j, 1), :] = cand[None, :]
            return 0

        jax.lax.fori_loop(0, kk, fbody, 0)

    topo_ref[0] = jax.lax.dot_general(
        erows_ref[0], Wp_ref[...], (((1,), (0,)), ((), ())),
        preferred_element_type=jnp.float32) + bp_ref[...][None, :]


def _sim_topk(norm, emb, Wp, bp, kk):
    b, s, e = emb.shape
    t = Wp.shape[1]
    grid = (b, s // _R)
    nd_t, idx_t, topo = pl.pallas_call(
        functools.partial(_sim_topk_kernel, kk=kk, s=s),
        grid=grid,
        in_specs=[
            pl.BlockSpec((1, _R, e), lambda bi, i: (bi, i, 0)),
            pl.BlockSpec((1, s, e), lambda bi, i: (bi, 0, 0)),
            pl.BlockSpec((1, _R, e), lambda bi, i: (bi, i, 0)),
            pl.BlockSpec((e, t), lambda bi, i: (0, 0)),
            pl.BlockSpec((t,), lambda bi, i: (0,)),
        ],
        out_specs=[
            pl.BlockSpec((1, kk, _R), lambda bi, i: (bi, 0, i)),
            pl.BlockSpec((1, kk, _R), lambda bi, i: (bi, 0, i)),
            pl.BlockSpec((1, _R, t), lambda bi, i: (bi, i, 0)),
        ],
        out_shape=[
            jax.ShapeDtypeStruct((b, kk, s), jnp.float32),
            jax.ShapeDtypeStruct((b, kk, s), jnp.int32),
            jax.ShapeDtypeStruct((b, s, t), jnp.float32),
        ],
        scratch_shapes=[
            pltpu.VMEM((_R, s), jnp.float32),
            pltpu.VMEM((_R, s), jnp.float32),
            pltpu.VMEM((_R, _D * 128), jnp.float32),
            pltpu.VMEM((_R, _D * 128), jnp.int32),
        ],
    )(norm, norm, emb, Wp, bp)
    return (jnp.swapaxes(nd_t, 1, 2), jnp.swapaxes(idx_t, 1, 2), topo)


_RC = 256


def _combine_mlp_kernel(topo_all_ref, topo_rows_ref, nd_ref, idx_ref,
                        W1_ref, b1_ref, W2_ref, b2_ref, gamma_ref, beta_ref,
                        W3_ref, b3_ref, W4_ref, b4_ref, out_ref, *, kk, s):
    nd = nd_ref[0]
    idx = idx_ref[0]
    mneg = jnp.max(-nd, axis=1, keepdims=True)
    ew = jnp.exp(-nd - mneg)
    w = ew / jnp.sum(ew, axis=1, keepdims=True)
    col = jax.lax.broadcasted_iota(jnp.int32, (_RC, s), 1)
    wb = w.astype(jnp.bfloat16)
    A = jnp.zeros((_RC, s), dtype=jnp.bfloat16)
    for k in range(kk):
        A = A + wb[:, k:k + 1] * (col == idx[:, k:k + 1]).astype(jnp.bfloat16)
    weighted = jax.lax.dot_general(A, topo_all_ref[0].astype(jnp.bfloat16),
                                   (((1,), (0,)), ((), ())),
                                   preferred_element_type=jnp.float32)
    comb = topo_rows_ref[0] + weighted
    h = jnp.maximum(jnp.dot(comb, W1_ref[...], preferred_element_type=jnp.float32)
                    + b1_ref[...][None, :], 0.0)
    h = jnp.dot(h, W2_ref[...], preferred_element_type=jnp.float32) + b2_ref[...][None, :]
    mu = jnp.mean(h, axis=-1, keepdims=True)
    var = jnp.mean((h - mu) ** 2, axis=-1, keepdims=True)
    hn = (h - mu) / jnp.sqrt(var + 1e-5) * gamma_ref[...][None, :] + beta_ref[...][None, :]
    g = jnp.maximum(jnp.dot(hn, W3_ref[...], preferred_element_type=jnp.float32)
                    + b3_ref[...][None, :], 0.0)
    out_ref[0] = jnp.dot(g, W4_ref[...], preferred_element_type=jnp.float32) + b4_ref[...][None, :]


def _combine_mlp(topo, nd, idx, W1, b1, W2, b2, gamma, beta, W3, b3, W4, b4, kk):
    b, s, t = topo.shape
    grid = (b, s // _RC)
    out = pl.pallas_call(
        functools.partial(_combine_mlp_kernel, kk=kk, s=s),
        grid=grid,
        in_specs=[
            pl.BlockSpec((1, s, t), lambda bi, i: (bi, 0, 0)),
            pl.BlockSpec((1, _RC, t), lambda bi, i: (bi, i, 0)),
            pl.BlockSpec((1, _RC, kk), lambda bi, i: (bi, i, 0)),
            pl.BlockSpec((1, _RC, kk), lambda bi, i: (bi, i, 0)),
            pl.BlockSpec(W1.shape, lambda bi, i: (0, 0)),
            pl.BlockSpec(b1.shape, lambda bi, i: (0,)),
            pl.BlockSpec(W2.shape, lambda bi, i: (0, 0)),
            pl.BlockSpec(b2.shape, lambda bi, i: (0,)),
            pl.BlockSpec(gamma.shape, lambda bi, i: (0,)),
            pl.BlockSpec(beta.shape, lambda bi, i: (0,)),
            pl.BlockSpec(W3.shape, lambda bi, i: (0, 0)),
            pl.BlockSpec(b3.shape, lambda bi, i: (0,)),
            pl.BlockSpec(W4.shape, lambda bi, i: (0, 0)),
            pl.BlockSpec(b4.shape, lambda bi, i: (0,)),
        ],
        out_specs=pl.BlockSpec((1, _RC, t), lambda bi, i: (bi, i, 0)),
        out_shape=jax.ShapeDtypeStruct((b, s, t), jnp.float32),
    )(topo, topo, nd, idx, W1, b1, W2, b2, gamma, beta, W3, b3, W4, b4)
    return out


def kernel(embeddings, Wp, bp, W1, b1, W2, b2, gamma, beta, W3, b3, W4, b4):
    b, s, e = embeddings.shape
    kk = max(1, min(32, s - 1))
    norm = embeddings / (jnp.linalg.norm(embeddings, axis=-1, keepdims=True) + 1e-8)
    nd, idx, topo = _sim_topk(norm, embeddings, Wp, bp, kk)
    p = _combine_mlp(topo, nd, idx, W1, b1, W2, b2, gamma, beta, W3, b3, W4, b4, kk)
    return (p, nd, idx)
